# baseline (speedup 1.0000x reference)
import functools

import jax
import jax.numpy as jnp
from jax import lax
from jax.experimental import pallas as pl
from jax.experimental.pallas import tpu as pltpu
from jax.experimental.pallas import tpu_sc as plsc


def _sc_gather2(idx_flat, table_c, table_t, n_rows, kc, kt, d, chunk,
                chunks_per_worker, nc, ns):
    per_worker = chunks_per_worker * chunk
    regions_per_batch = (kc + kt) // per_worker
    ctx_regions = kc // per_worker
    mesh = plsc.VectorSubcoreMesh(core_axis_name="c", subcore_axis_name="s")

    @functools.partial(
        pl.kernel,
        mesh=mesh,
        out_type=jax.ShapeDtypeStruct((n_rows, d), jnp.float32),
        scratch_types=[
            pltpu.VMEM((per_worker,), jnp.int32),
            pltpu.VMEM((chunk, d), jnp.float32),
            pltpu.SemaphoreType.DMA,
        ],
    )
    def gather_k(idx_hbm, tc_hbm, tt_hbm, out_hbm, idx_v, rows_v, sem):
        wid = lax.axis_index("s") * nc + lax.axis_index("c")
        base = wid * per_worker
        pltpu.sync_copy(idx_hbm.at[pl.ds(base, per_worker)], idx_v)
        is_ctx = lax.rem(wid, regions_per_batch) < ctx_regions

        @pl.when(is_ctx)
        def _():
            for j in range(chunks_per_worker):
                pltpu.async_copy(
                    tc_hbm.at[idx_v.at[pl.ds(j * chunk, chunk)]], rows_v, sem
                ).wait()
                pltpu.sync_copy(rows_v, out_hbm.at[pl.ds(base + j * chunk, chunk)])

        @pl.when(jnp.logical_not(is_ctx))
        def _():
            for j in range(chunks_per_worker):
                pltpu.async_copy(
                    tt_hbm.at[idx_v.at[pl.ds(j * chunk, chunk)]], rows_v, sem
                ).wait()
                pltpu.sync_copy(rows_v, out_hbm.at[pl.ds(base + j * chunk, chunk)])

    return gather_k(idx_flat, table_c, table_t)


def _tc_matmul(hidden_states, W, n_ctx_blocks, rb):
    B, Kc, E = hidden_states.shape
    D = W.shape[1]

    def body(hs_ref, w_ref, out_ref):
        out_ref[0] = jax.lax.dot_general(
            hs_ref[0].astype(jnp.bfloat16), w_ref[...].astype(jnp.bfloat16),
            (((1,), (0,)), ((), ())),
            preferred_element_type=jnp.float32,
        ).astype(jnp.bfloat16)

    return pl.pallas_call(
        body,
        grid=(B, n_ctx_blocks),
        in_specs=[
            pl.BlockSpec((1, rb, E), lambda i, r: (i, r, 0)),
            pl.BlockSpec((E, D), lambda i, r: (0, 0)),
        ],
        out_specs=pl.BlockSpec((1, rb, D), lambda i, r: (i, r, 0)),
        out_shape=jax.ShapeDtypeStruct((B, Kc, D), jnp.bfloat16),
    )(hidden_states, W)


def _tc_add(mm, pos_all, n_ctx_blocks, rb):
    B, Kc, D = mm.shape
    K_total = pos_all.shape[1]

    def body(mm_ref, pos_ref, out_ref):
        out_ref[0] = mm_ref[0].astype(jnp.float32) + pos_ref[0]

    return pl.pallas_call(
        body,
        grid=(B, n_ctx_blocks),
        in_specs=[
            pl.BlockSpec((1, rb, D), lambda i, r: (i, r, 0)),
            pl.BlockSpec((1, rb, D), lambda i, r: (i, r, 0)),
        ],
        out_specs=pl.BlockSpec((1, rb, D), lambda i, r: (i, r, 0)),
        out_shape=jax.ShapeDtypeStruct((B, K_total, D), jnp.float32),
        input_output_aliases={1: 0},
    )(mm, pos_all)


def kernel(hidden_states, context_mask, target_mask, mask_index, W, b, mask_token, pos_embed):
    B, Kc, E = hidden_states.shape
    Kt = target_mask.shape[1]
    D = W.shape[1]
    K_total = Kc + Kt

    masks = jnp.concatenate([context_mask, target_mask], axis=1)

    table_c = pos_embed + b[None, :]
    table_t = pos_embed + mask_token[0]

    nc, ns = 2, 16
    nw = nc * ns
    chunk = 128
    n_rows = B * K_total
    n_ctx_rows = B * Kc
    chunks_per_worker = n_rows // (nw * chunk)

    pos_all = _sc_gather2(masks.reshape(n_rows), table_c, table_t, n_rows,
                          Kc, Kt, D, chunk, chunks_per_worker, nc, ns)
    pos_all = pos_all.reshape(B, K_total, D)

    rb = 1728
    n_ctx_blocks = Kc // rb
    mm = _tc_matmul(hidden_states, W, n_ctx_blocks, rb)
    embeddings = _tc_add(mm, pos_all, n_ctx_blocks, rb)
    return (embeddings, masks)
